# baseline (speedup 1.0000x reference)
import functools

import jax
import jax.numpy as jnp
from jax import lax
from jax.experimental import pallas as pl
from jax.experimental.pallas import tpu as pltpu
from jax.experimental.pallas import tpu_sc as plsc

N = 10000
E = 320000
D = 128
H = 128
HP = 144
NGO = 2827
GO_PAD = 2944
NB_RELS = 9

NC = 2
NS = 16
NW = NC * NS
EPT = E // NW
K = 80
NCH = EPT // K
RPT = 624
RPT_LAST = N - (NS - 1) * RPT

BN_FRONT = 2000
BN_EPI = 400


def _front_body(x_ref, w1_ref, b1_ref, wg_ref, attn2_ref, feat_ref, eler_ref):
    h = jnp.dot(x_ref[...], w1_ref[...], preferred_element_type=jnp.float32)
    h = jnp.maximum(h + b1_ref[...], 0.0)
    feat = jnp.dot(h, wg_ref[...], preferred_element_type=jnp.float32)
    feat_ref[...] = jnp.concatenate(
        [feat, jnp.zeros((feat.shape[0], HP - H), jnp.float32)], axis=1)
    eler_ref[...] = jnp.dot(feat, attn2_ref[...],
                            preferred_element_type=jnp.float32)


def _front(x, w1, b1_2d, wg, attn2):
    grid = (N // BN_FRONT,)
    return pl.pallas_call(
        _front_body,
        grid=grid,
        in_specs=[
            pl.BlockSpec((BN_FRONT, D), lambda i: (i, 0)),
            pl.BlockSpec((D, H), lambda i: (0, 0)),
            pl.BlockSpec((1, H), lambda i: (0, 0)),
            pl.BlockSpec((H, H), lambda i: (0, 0)),
            pl.BlockSpec((H, 2), lambda i: (0, 0)),
        ],
        out_specs=[
            pl.BlockSpec((BN_FRONT, HP), lambda i: (i, 0)),
            pl.BlockSpec((BN_FRONT, 2), lambda i: (i, 0)),
        ],
        out_shape=[
            jax.ShapeDtypeStruct((N, HP), jnp.float32),
            jax.ShapeDtypeStruct((N, 2), jnp.float32),
        ],
    )(x, w1, b1_2d, wg, attn2)


def _sc_ee(src_e, dst_e, eler_flat):
    mesh = plsc.VectorSubcoreMesh(core_axis_name="c", subcore_axis_name="s",
                                  num_cores=NC)

    @functools.partial(
        pl.kernel,
        out_type=jax.ShapeDtypeStruct((E,), jnp.float32),
        mesh=mesh,
        compiler_params=pltpu.CompilerParams(needs_layout_passes=False,
                                             use_tc_tiling_on_sc=False),
        scratch_types=[
            pltpu.VMEM((2 * N,), jnp.float32),
            pltpu.VMEM((EPT,), jnp.int32),
            pltpu.VMEM((EPT,), jnp.int32),
            pltpu.VMEM((EPT,), jnp.float32),
        ],
    )
    def body(src_hbm, dst_hbm, eler_hbm, ee_hbm, eler_v, src_v, dst_v, ee_v):
        c = lax.axis_index("c")
        s = lax.axis_index("s")
        wid = s * NC + c
        base_e = wid * EPT
        pltpu.sync_copy(src_hbm.at[pl.ds(base_e, EPT)], src_v)
        pltpu.sync_copy(dst_hbm.at[pl.ds(base_e, EPT)], dst_v)
        pltpu.sync_copy(eler_hbm, eler_v)

        def sbody(i, carry):
            sl = pl.ds(i * 16, 16)
            sv = src_v[sl]
            dv = dst_v[sl]
            elv = plsc.load_gather(eler_v, [sv * 2])
            erv = plsc.load_gather(eler_v, [dv * 2 + 1])
            e = elv + erv
            e = jnp.where(e >= 0.0, e, e * jnp.float32(0.2))
            ee_v[sl] = jnp.exp(e)
            return carry

        lax.fori_loop(0, EPT // 16, sbody, 0)
        pltpu.sync_copy(ee_v, ee_hbm.at[pl.ds(base_e, EPT)])

    return body(src_e, dst_e, eler_flat)


BLK = 2000
NBLK = EPT // BLK
CPB = BLK // K


def _sc_rows(src_e, dst_e, ee, feat):
    mesh = plsc.VectorSubcoreMesh(core_axis_name="c", subcore_axis_name="s",
                                  num_cores=NC)

    @functools.partial(
        pl.kernel,
        out_type=jax.ShapeDtypeStruct((NC, N, HP), jnp.float32),
        mesh=mesh,
        compiler_params=pltpu.CompilerParams(needs_layout_passes=False,
                                             use_tc_tiling_on_sc=False),
        scratch_types=[
            pltpu.VMEM((BLK,), jnp.int32),
            pltpu.VMEM((BLK,), jnp.int32),
            pltpu.VMEM((BLK,), jnp.float32),
            pltpu.VMEM((BLK,), jnp.int32),
            pltpu.VMEM((BLK,), jnp.int32),
            pltpu.VMEM((BLK,), jnp.float32),
            pltpu.VMEM((K, HP), jnp.float32),
            pltpu.VMEM((K, HP), jnp.float32),
            pltpu.VMEM((K,), jnp.int32),
            pltpu.VMEM((K,), jnp.int32),
            pltpu.VMEM_SHARED((N, HP), jnp.float32),
            pltpu.SemaphoreType.DMA,
            pltpu.SemaphoreType.DMA,
            pltpu.SemaphoreType.DMA,
            pltpu.SemaphoreType.DMA,
            pltpu.SemaphoreType.DMA,
            pltpu.SemaphoreType.DMA,
        ],
    )
    def body(src_hbm, dst_hbm, ee_hbm, feat_hbm, outu_hbm,
             srcb0, dstb0, eeb0, srcb1, dstb1, eeb1, rows0, rows1,
             sidx0, sidx1, out_sp, bsem0, bsem1, gsem0, gsem1, ssem0, ssem1):
        c = lax.axis_index("c")
        s = lax.axis_index("s")
        wid = s * NC + c
        base_e = wid * EPT
        row0 = s * RPT

        blks = ((srcb0, dstb0, eeb0, bsem0), (srcb1, dstb1, eeb1, bsem1))
        sets = ((rows0, sidx0, gsem0, ssem0), (rows1, sidx1, gsem1, ssem1))

        zeros16 = jnp.zeros((16,), jnp.float32)

        def zrow(k, carry):
            for j in range(HP // 16):
                rows0[k, pl.ds(j * 16, 16)] = zeros16
            return carry

        lax.fori_loop(0, K, zrow, 0)

        def _zero_slice(nrows):
            full = nrows // K
            for t in range(full):
                pltpu.sync_copy(rows0, out_sp.at[pl.ds(row0 + t * K, K)])
            rem = nrows - full * K
            if rem:
                pltpu.sync_copy(rows0.at[pl.ds(0, rem)],
                                out_sp.at[pl.ds(row0 + full * K, rem)])

        @pl.when(s == NS - 1)
        def _():
            _zero_slice(RPT_LAST)

        @pl.when(s != NS - 1)
        def _():
            _zero_slice(RPT)

        plsc.subcore_barrier()

        lane = lax.iota(jnp.int32, 16)
        is0 = lane == 0

        def blk_copies(b, bb):
            eb = base_e + b * BLK
            return (pltpu.make_async_copy(src_hbm.at[pl.ds(eb, BLK)], bb[0],
                                          bb[3]),
                    pltpu.make_async_copy(dst_hbm.at[pl.ds(eb, BLK)], bb[1],
                                          bb[3]),
                    pltpu.make_async_copy(ee_hbm.at[pl.ds(eb, BLK)], bb[2],
                                          bb[3]))

        def gather_local(bb, l, st):
            return pltpu.make_async_copy(
                feat_hbm.at[bb[0].at[pl.ds(l * K, K)]], st[0], st[2])

        def scatter_copy(st):
            return pltpu.make_async_copy(st[0], out_sp.at[st[1]], st[3])

        def scale_snap(bb, l, st):
            rows, sidx = st[0], st[1]
            eo = l * K

            def sbody(kk, carry):
                eev = bb[2][pl.ds(eo + kk * 16, 16)]
                for i in range(16):
                    sval = eev[i]
                    r = kk * 16 + i
                    for j in range(H // 16):
                        sl = pl.ds(j * 16, 16)
                        rows[r, sl] = rows[r, sl] * sval
                    rows[r, pl.ds(H, 16)] = jnp.where(is0, sval, 0.0)
                return carry

            lax.fori_loop(0, K // 16, sbody, 0)
            for j in range(K // 16):
                sl = pl.ds(j * 16, 16)
                sidx[sl] = bb[1][pl.ds(eo + j * 16, 16)]

        def chunk_local(l, cur, nxt, bb, tail, skip_sw):
            gather_local(bb, l, cur).wait()
            if not tail:
                if skip_sw is None:
                    scatter_copy(nxt).wait()
                else:
                    @pl.when(skip_sw)
                    def _():
                        scatter_copy(nxt).wait()
                gather_local(bb, l + 1, nxt).start()
            scale_snap(bb, l, cur)
            pltpu.async_copy(cur[0], out_sp.at[cur[1]], cur[3], add=True)

        for cpy in blk_copies(0, blks[0]):
            cpy.start()
        for cpy in blk_copies(1, blks[1]):
            cpy.start()

        for b in range(NBLK):
            bb = blks[b % 2]
            p = b % 2
            cu, nx = sets[p], sets[1 - p]
            if b > 0:
                scatter_copy(sets[0]).wait()
                scatter_copy(sets[1]).wait()
            for cpy in blk_copies(b, bb):
                cpy.wait()
            gather_local(bb, 0, cu).start()

            def pairs(t, carry, bb=bb, cu=cu, nx=nx, b=b):
                chunk_local(2 * t, cu, nx, bb, False, t > 0)
                chunk_local(2 * t + 1, nx, cu, bb, False, None)
                return carry

            lax.fori_loop(0, (CPB - 1) // 2, pairs, 0)
            chunk_local(CPB - 1, cu, nx, bb, True, None)
            if b + 2 < NBLK:
                for cpy in blk_copies(b + 2, blks[b % 2]):
                    cpy.start()

        scatter_copy(sets[0]).wait()
        scatter_copy(sets[1]).wait()

        plsc.subcore_barrier()

        @pl.when(s == NS - 1)
        def _():
            pltpu.sync_copy(out_sp.at[pl.ds(row0, RPT_LAST)],
                            outu_hbm.at[c, pl.ds(row0, RPT_LAST)])

        @pl.when(s != NS - 1)
        def _():
            pltpu.sync_copy(out_sp.at[pl.ds(row0, RPT)],
                            outu_hbm.at[c, pl.ds(row0, RPT)])

    return body(src_e, dst_e, ee, feat)


def _norm_body(outu_ref, bias_ref, o_ref):
    ou = outu_ref[0] + outu_ref[1]
    den = ou[:, H:H + 1]
    o_ref[...] = ou[:, :H] / (den + 1e-16) + bias_ref[...]


def _norm(outu, bias_2d):
    grid = (N // BN_FRONT,)
    return pl.pallas_call(
        _norm_body,
        grid=grid,
        in_specs=[
            pl.BlockSpec((NC, BN_FRONT, HP), lambda i: (0, i, 0)),
            pl.BlockSpec((1, H), lambda i: (0, 0)),
        ],
        out_specs=pl.BlockSpec((BN_FRONT, H), lambda i: (i, 0)),
        out_shape=jax.ShapeDtypeStruct((N, H), jnp.float32),
    )(outu, bias_2d)


BGO = 128


def _epi_body(o_ref, go_ref, rad_ref, rel_ref, out_ref):
    gohf = go_ref[...] + rel_ref[...][NB_RELS][None, :]
    sc = lax.dot_general(gohf, o_ref[...], (((1,), (1,)), ((), ())),
                         preferred_element_type=jnp.float32)
    sc = sc + jnp.abs(rad_ref[...])
    out_ref[...] = jax.nn.sigmoid(sc)


def _epi(o, go_embed_w, go_rad_w, rel_embed_w):
    grid = (pl.cdiv(NGO, BGO),)
    return pl.pallas_call(
        _epi_body,
        grid=grid,
        in_specs=[
            pl.BlockSpec((N, H), lambda i: (0, 0)),
            pl.BlockSpec((BGO, H), lambda i: (i, 0)),
            pl.BlockSpec((BGO, 1), lambda i: (i, 0)),
            pl.BlockSpec((NB_RELS + 1, H), lambda i: (0, 0)),
        ],
        out_specs=pl.BlockSpec((BGO, N), lambda i: (i, 0)),
        out_shape=jax.ShapeDtypeStruct((NGO, N), jnp.float32),
    )(o, go_embed_w, go_rad_w, rel_embed_w)


def kernel(x, edge_index, W1, b1, Wg, attn_l, attn_r, bias_g, go_embed_w,
           go_rad_w, rel_embed_w):
    attn2 = jnp.stack([attn_l, attn_r], axis=1)
    feat, el_er = _front(x, W1, b1.reshape(1, -1), Wg, attn2)
    ei = edge_index.astype(jnp.int32)
    ee = _sc_ee(ei[0], ei[1], el_er.reshape(-1))
    outu = _sc_rows(ei[0], ei[1], ee, feat)
    o = _norm(outu, bias_g.reshape(1, -1))
    logits_t = _epi(o, go_embed_w, go_rad_w, rel_embed_w)
    return logits_t.T


# speedup vs baseline: 1.1085x; 1.1085x over previous
import functools

import jax
import jax.numpy as jnp
from jax import lax
from jax.experimental import pallas as pl
from jax.experimental.pallas import tpu as pltpu
from jax.experimental.pallas import tpu_sc as plsc

N = 10000
E = 320000
D = 128
H = 128
HP = 144
NGO = 2827
GO_PAD = 2944
NB_RELS = 9

NC = 2
NS = 16
NW = NC * NS
EPT = E // NW
K = 80
NCH = EPT // K
RPT = 624
RPT_LAST = N - (NS - 1) * RPT

BN_FRONT = 2000
BN_EPI = 400


def _front_body(x_ref, w1_ref, b1_ref, wg_ref, attn2_ref, feat_ref, eler_ref):
    h = jnp.dot(x_ref[...], w1_ref[...], preferred_element_type=jnp.float32)
    h = jnp.maximum(h + b1_ref[...], 0.0)
    feat = jnp.dot(h, wg_ref[...], preferred_element_type=jnp.float32)
    feat_ref[...] = jnp.concatenate(
        [feat, jnp.zeros((feat.shape[0], HP - H), jnp.float32)], axis=1)
    eler_ref[...] = jnp.dot(feat, attn2_ref[...],
                            preferred_element_type=jnp.float32)


def _front(x, w1, b1_2d, wg, attn2):
    grid = (N // BN_FRONT,)
    return pl.pallas_call(
        _front_body,
        grid=grid,
        in_specs=[
            pl.BlockSpec((BN_FRONT, D), lambda i: (i, 0)),
            pl.BlockSpec((D, H), lambda i: (0, 0)),
            pl.BlockSpec((1, H), lambda i: (0, 0)),
            pl.BlockSpec((H, H), lambda i: (0, 0)),
            pl.BlockSpec((H, 2), lambda i: (0, 0)),
        ],
        out_specs=[
            pl.BlockSpec((BN_FRONT, HP), lambda i: (i, 0)),
            pl.BlockSpec((BN_FRONT, 2), lambda i: (i, 0)),
        ],
        out_shape=[
            jax.ShapeDtypeStruct((N, HP), jnp.float32),
            jax.ShapeDtypeStruct((N, 2), jnp.float32),
        ],
    )(x, w1, b1_2d, wg, attn2)


def _sc_ee(src_e, dst_e, eler_flat):
    mesh = plsc.VectorSubcoreMesh(core_axis_name="c", subcore_axis_name="s",
                                  num_cores=NC)

    @functools.partial(
        pl.kernel,
        out_type=jax.ShapeDtypeStruct((E,), jnp.float32),
        mesh=mesh,
        compiler_params=pltpu.CompilerParams(needs_layout_passes=False,
                                             use_tc_tiling_on_sc=False),
        scratch_types=[
            pltpu.VMEM((2 * N,), jnp.float32),
            pltpu.VMEM((EPT,), jnp.int32),
            pltpu.VMEM((EPT,), jnp.int32),
            pltpu.VMEM((EPT,), jnp.float32),
        ],
    )
    def body(src_hbm, dst_hbm, eler_hbm, ee_hbm, eler_v, src_v, dst_v, ee_v):
        c = lax.axis_index("c")
        s = lax.axis_index("s")
        wid = s * NC + c
        base_e = wid * EPT
        pltpu.sync_copy(src_hbm.at[pl.ds(base_e, EPT)], src_v)
        pltpu.sync_copy(dst_hbm.at[pl.ds(base_e, EPT)], dst_v)
        pltpu.sync_copy(eler_hbm, eler_v)

        def sbody(i, carry):
            sl = pl.ds(i * 16, 16)
            sv = src_v[sl]
            dv = dst_v[sl]
            elv = plsc.load_gather(eler_v, [sv * 2])
            erv = plsc.load_gather(eler_v, [dv * 2 + 1])
            e = elv + erv
            e = jnp.where(e >= 0.0, e, e * jnp.float32(0.2))
            ee_v[sl] = jnp.exp(e)
            return carry

        lax.fori_loop(0, EPT // 16, sbody, 0)
        pltpu.sync_copy(ee_v, ee_hbm.at[pl.ds(base_e, EPT)])

    return body(src_e, dst_e, eler_flat)


def _sc_rows(src_e, dst_e, ee, feat):
    mesh = plsc.VectorSubcoreMesh(core_axis_name="c", subcore_axis_name="s",
                                  num_cores=NC)

    @functools.partial(
        pl.kernel,
        out_type=jax.ShapeDtypeStruct((NC, N, HP), jnp.float32),
        mesh=mesh,
        compiler_params=pltpu.CompilerParams(needs_layout_passes=False,
                                             use_tc_tiling_on_sc=False),
        scratch_types=[
            pltpu.VMEM((K,), jnp.int32),
            pltpu.VMEM((K,), jnp.int32),
            pltpu.VMEM((K,), jnp.float32),
            pltpu.VMEM((K, HP), jnp.float32),
            pltpu.VMEM((K,), jnp.int32),
            pltpu.VMEM((K,), jnp.int32),
            pltpu.VMEM((K,), jnp.float32),
            pltpu.VMEM((K, HP), jnp.float32),
            pltpu.VMEM((K,), jnp.int32),
            pltpu.VMEM((K,), jnp.int32),
            pltpu.VMEM((K,), jnp.float32),
            pltpu.VMEM((K, HP), jnp.float32),
            pltpu.VMEM((K,), jnp.int32),
            pltpu.VMEM((K,), jnp.int32),
            pltpu.VMEM((K,), jnp.int32),
            pltpu.VMEM_SHARED((N, HP), jnp.float32),
            pltpu.SemaphoreType.DMA,
            pltpu.SemaphoreType.DMA,
            pltpu.SemaphoreType.DMA,
            pltpu.SemaphoreType.DMA,
            pltpu.SemaphoreType.DMA,
            pltpu.SemaphoreType.DMA,
            pltpu.SemaphoreType.DMA,
            pltpu.SemaphoreType.DMA,
            pltpu.SemaphoreType.DMA,
        ],
    )
    def body(src_hbm, dst_hbm, ee_hbm, feat_hbm, outu_hbm,
             srci0, dsti0, eei0, rows0, srci1, dsti1, eei1, rows1,
             srci2, dsti2, eei2, rows2, sidx0, sidx1, sidx2, out_sp,
             isem0, isem1, isem2, gsem0, gsem1, gsem2, ssem0, ssem1, ssem2):
        c = lax.axis_index("c")
        s = lax.axis_index("s")
        wid = s * NC + c
        base_e = wid * EPT
        row0 = s * RPT

        sets = ((srci0, dsti0, eei0, rows0, isem0, gsem0, sidx0, ssem0),
                (srci1, dsti1, eei1, rows1, isem1, gsem1, sidx1, ssem1),
                (srci2, dsti2, eei2, rows2, isem2, gsem2, sidx2, ssem2))

        zeros16 = jnp.zeros((16,), jnp.float32)

        def zrow(k, carry):
            for j in range(HP // 16):
                rows0[k, pl.ds(j * 16, 16)] = zeros16
            return carry

        lax.fori_loop(0, K, zrow, 0)

        def _zero_slice(nrows):
            full = nrows // K
            for t in range(full):
                pltpu.sync_copy(rows0, out_sp.at[pl.ds(row0 + t * K, K)])
            rem = nrows - full * K
            if rem:
                pltpu.sync_copy(rows0.at[pl.ds(0, rem)],
                                out_sp.at[pl.ds(row0 + full * K, rem)])

        @pl.when(s == NS - 1)
        def _():
            _zero_slice(RPT_LAST)

        @pl.when(s != NS - 1)
        def _():
            _zero_slice(RPT)

        plsc.subcore_barrier()

        lane = lax.iota(jnp.int32, 16)
        is0 = lane == 0

        def idx_copies(ch, st):
            eb = base_e + ch * K
            return (pltpu.make_async_copy(src_hbm.at[pl.ds(eb, K)], st[0],
                                          st[4]),
                    pltpu.make_async_copy(dst_hbm.at[pl.ds(eb, K)], st[1],
                                          st[4]),
                    pltpu.make_async_copy(ee_hbm.at[pl.ds(eb, K)], st[2],
                                          st[4]))

        def gather_copy(st):
            return pltpu.make_async_copy(feat_hbm.at[st[0]], st[3], st[5])

        for t in range(3):
            for cpy in idx_copies(t, sets[t]):
                cpy.start()
        for cpy in idx_copies(0, sets[0]):
            cpy.wait()
        gather_copy(sets[0]).start()
        for cpy in idx_copies(1, sets[1]):
            cpy.wait()
        gather_copy(sets[1]).start()

        def scale(st):
            rows, eei = st[3], st[2]

            def sbody(kk, carry):
                eev = eei[pl.ds(kk * 16, 16)]
                for i in range(16):
                    sval = eev[i]
                    r = kk * 16 + i
                    for j in range(H // 16):
                        sl = pl.ds(j * 16, 16)
                        rows[r, sl] = rows[r, sl] * sval
                    rows[r, pl.ds(H, 16)] = jnp.where(is0, sval, 0.0)
                return carry

            lax.fori_loop(0, K // 16, sbody, 0)

        def scatter_copy(st):
            return pltpu.make_async_copy(st[3], out_sp.at[st[6]], st[7])

        def chunk_step(ch, cur, nxt):
            gather_copy(cur).wait()

            @pl.when(ch + 2 < NCH)
            def _():
                for cpy in idx_copies(ch + 2, nxt):
                    cpy.wait()

                @pl.when(ch >= 1)
                def _():
                    scatter_copy(nxt).wait()

                gather_copy(nxt).start()

            scale(cur)
            for j in range(K // 16):
                sl = pl.ds(j * 16, 16)
                cur[6][sl] = cur[1][sl]
            pltpu.async_copy(cur[3], out_sp.at[cur[6]], cur[7], add=True)

            @pl.when(ch + 3 < NCH)
            def _():
                for cpy in idx_copies(ch + 3, cur):
                    cpy.start()

        def trip(i, carry):
            for t in range(3):
                ch = 3 * i + t

                @pl.when(ch < NCH)
                def _(ch=ch, t=t):
                    chunk_step(ch, sets[t], sets[(t + 2) % 3])

            return carry

        lax.fori_loop(0, (NCH + 2) // 3, trip, 0)

        scatter_copy(sets[0]).wait()
        scatter_copy(sets[1]).wait()
        scatter_copy(sets[2]).wait()

        plsc.subcore_barrier()

        @pl.when(s == NS - 1)
        def _():
            pltpu.sync_copy(out_sp.at[pl.ds(row0, RPT_LAST)],
                            outu_hbm.at[c, pl.ds(row0, RPT_LAST)])

        @pl.when(s != NS - 1)
        def _():
            pltpu.sync_copy(out_sp.at[pl.ds(row0, RPT)],
                            outu_hbm.at[c, pl.ds(row0, RPT)])

    return body(src_e, dst_e, ee, feat)


def _norm_body(outu_ref, bias_ref, o_ref):
    ou = outu_ref[0] + outu_ref[1]
    den = ou[:, H:H + 1]
    o_ref[...] = ou[:, :H] / (den + 1e-16) + bias_ref[...]


def _norm(outu, bias_2d):
    grid = (N // BN_FRONT,)
    return pl.pallas_call(
        _norm_body,
        grid=grid,
        in_specs=[
            pl.BlockSpec((NC, BN_FRONT, HP), lambda i: (0, i, 0)),
            pl.BlockSpec((1, H), lambda i: (0, 0)),
        ],
        out_specs=pl.BlockSpec((BN_FRONT, H), lambda i: (i, 0)),
        out_shape=jax.ShapeDtypeStruct((N, H), jnp.float32),
    )(outu, bias_2d)


BGO = 128


def _epi_body(o_ref, go_ref, rad_ref, rel_ref, out_ref):
    gohf = go_ref[...] + rel_ref[...][NB_RELS][None, :]
    sc = lax.dot_general(gohf, o_ref[...], (((1,), (1,)), ((), ())),
                         preferred_element_type=jnp.float32)
    sc = sc + jnp.abs(rad_ref[...])
    out_ref[...] = jax.nn.sigmoid(sc)


def _epi(o, go_embed_w, go_rad_w, rel_embed_w):
    grid = (pl.cdiv(NGO, BGO),)
    return pl.pallas_call(
        _epi_body,
        grid=grid,
        in_specs=[
            pl.BlockSpec((N, H), lambda i: (0, 0)),
            pl.BlockSpec((BGO, H), lambda i: (i, 0)),
            pl.BlockSpec((BGO, 1), lambda i: (i, 0)),
            pl.BlockSpec((NB_RELS + 1, H), lambda i: (0, 0)),
        ],
        out_specs=pl.BlockSpec((BGO, N), lambda i: (i, 0)),
        out_shape=jax.ShapeDtypeStruct((NGO, N), jnp.float32),
    )(o, go_embed_w, go_rad_w, rel_embed_w)


def kernel(x, edge_index, W1, b1, Wg, attn_l, attn_r, bias_g, go_embed_w,
           go_rad_w, rel_embed_w):
    attn2 = jnp.stack([attn_l, attn_r], axis=1)
    feat, el_er = _front(x, W1, b1.reshape(1, -1), Wg, attn2)
    ei = edge_index.astype(jnp.int32)
    ee = _sc_ee(ei[0], ei[1], el_er.reshape(-1))
    outu = _sc_rows(ei[0], ei[1], ee, feat)
    o = _norm(outu, bias_g.reshape(1, -1))
    logits_t = _epi(o, go_embed_w, go_rad_w, rel_embed_w)
    return logits_t.T
